# baseline (speedup 1.0000x reference)
import jax
import jax.numpy as jnp
from jax import lax
from jax.experimental import pallas as pl
from jax.experimental.pallas import tpu as pltpu
from jax.experimental.pallas import tpu_sc as plsc

_NC, _NS, _L = 2, 16, 16
_NW = _NC * _NS
_B = 16384
_D = 32
_BPW = _B // _NW
_G = _BPW // _L


def _body(uids, iids, uemb, iemb, out, uid_v, iid_v, urow_v, irow_v, out_v,
          sem_u, sem_i):
    wid = lax.axis_index("s") * _NC + lax.axis_index("c")
    base = wid * _BPW
    pltpu.sync_copy(uids.at[pl.ds(base, _BPW)], uid_v)
    pltpu.sync_copy(iids.at[pl.ds(base, _BPW)], iid_v)
    cu = pltpu.async_copy(uemb.at[uid_v], urow_v, sem_u)
    ci = pltpu.async_copy(iemb.at[iid_v], irow_v, sem_i)
    cu.wait()
    ci.wait()

    iota = lax.iota(jnp.int32, _L)

    def group(g, carry):
        rows = g * _L + iota
        acc = jnp.zeros((_L,), jnp.float32)
        for c in range(_D):
            cols = jnp.full((_L,), c, jnp.int32)
            uv = plsc.load_gather(urow_v, [rows, cols])
            iv = plsc.load_gather(irow_v, [rows, cols])
            acc = acc + uv * iv
        out_v[pl.ds(g * _L, _L)] = acc
        return carry

    lax.fori_loop(0, _G, group, 0)
    pltpu.sync_copy(out_v, out.at[pl.ds(base, _BPW)])


def kernel(user_ids, item_ids, user_emb, item_emb):
    mesh = plsc.VectorSubcoreMesh(
        core_axis_name="c", subcore_axis_name="s",
        num_cores=_NC, num_subcores=_NS)
    f = pl.kernel(
        _body,
        out_type=jax.ShapeDtypeStruct((_B,), jnp.float32),
        mesh=mesh,
        scratch_types=[
            pltpu.VMEM((_BPW,), jnp.int32),
            pltpu.VMEM((_BPW,), jnp.int32),
            pltpu.VMEM((_BPW, _D), jnp.float32),
            pltpu.VMEM((_BPW, _D), jnp.float32),
            pltpu.VMEM((_BPW,), jnp.float32),
            pltpu.SemaphoreType.DMA,
            pltpu.SemaphoreType.DMA,
        ],
        compiler_params=pltpu.CompilerParams(
            needs_layout_passes=False, use_tc_tiling_on_sc=False),
    )
    return f(user_ids, item_ids, user_emb, item_emb)
